# Initial kernel scaffold; baseline (speedup 1.0000x reference)
import jax
import jax.numpy as jnp
from jax.experimental import pallas as pl


def kernel(x, W1, b1, W2, b2, batch):
    raise NotImplementedError("write your pallas kernel here")


# speedup vs baseline: 11.1430x; 11.1430x over previous
import functools

import jax
import jax.numpy as jnp
from jax.experimental import pallas as pl
from jax.experimental.pallas import tpu as pltpu

_NEG = float("-inf")


def _body(x_ref, w1_ref, b1_ref, w2_ref, batch_ref, out_ref, rmax_ref, den_ref,
          *, nseg, blk):
    i = pl.program_id(0)
    nb = pl.num_programs(0)

    @pl.when(i == 0)
    def _init():
        rmax_ref[...] = jnp.full((nseg, 1), _NEG, jnp.float32)
        den_ref[...] = jnp.zeros((nseg, 1), jnp.float32)
        out_ref[...] = jnp.zeros_like(out_ref)

    x = x_ref[...]
    h = jnp.dot(x, w1_ref[...], preferred_element_type=jnp.float32)
    h = h + b1_ref[...]
    h = h * jax.nn.sigmoid(h)
    lt = jnp.sum(h * w2_ref[...], axis=1).reshape(1, blk)

    bt = batch_ref[0]
    seg = jax.lax.broadcasted_iota(jnp.int32, (nseg, 1), 0)
    oh = bt == seg

    bmax = jnp.max(jnp.where(oh, lt, _NEG), axis=1, keepdims=True)
    rm = rmax_ref[...]
    nm = jnp.maximum(rm, bmax)
    scale = jnp.where(nm == rm, 1.0, jnp.exp(rm - nm))
    nm_b = jnp.max(jnp.where(oh, nm, _NEG), axis=0, keepdims=True)
    e = jnp.exp(lt - nm_b)

    ohf = oh.astype(jnp.float32)
    den_ref[...] = den_ref[...] * scale + jnp.sum(ohf * e, axis=1, keepdims=True)
    xe = x * e.reshape(blk, 1)
    out_ref[...] = out_ref[...] * scale + jnp.dot(
        ohf, xe, preferred_element_type=jnp.float32)
    rmax_ref[...] = nm

    @pl.when(i == nb - 1)
    def _fin():
        out_ref[...] = out_ref[...] / (den_ref[...] + 1e-16)


def kernel(x, W1, b1, W2, b2, batch):
    n, d = x.shape
    h = W1.shape[1]
    nseg = 512
    blk = 2000 if n % 2000 == 0 else n
    nb = n // blk

    batch3 = batch.astype(jnp.int32).reshape(nb, 1, blk)
    b1r = b1.reshape(1, h)
    w2r = W2.reshape(1, h)

    return pl.pallas_call(
        functools.partial(_body, nseg=nseg, blk=blk),
        grid=(nb,),
        in_specs=[
            pl.BlockSpec((blk, d), lambda i: (i, 0)),
            pl.BlockSpec((d, h), lambda i: (0, 0)),
            pl.BlockSpec((1, h), lambda i: (0, 0)),
            pl.BlockSpec((1, h), lambda i: (0, 0)),
            pl.BlockSpec((1, 1, blk), lambda i: (i, 0, 0)),
        ],
        out_specs=pl.BlockSpec((nseg, d), lambda i: (0, 0)),
        out_shape=jax.ShapeDtypeStruct((nseg, d), jnp.float32),
        scratch_shapes=[
            pltpu.VMEM((nseg, 1), jnp.float32),
            pltpu.VMEM((nseg, 1), jnp.float32),
        ],
    )(x, W1, b1r, w2r, batch3)
